# baseline (speedup 1.0000x reference)
import functools

import jax
import jax.numpy as jnp
from jax import lax
from jax.experimental import pallas as pl
from jax.experimental.pallas import tpu as pltpu
from jax.experimental.pallas import tpu_sc as plsc

NC = 2
NS = 16
NW = NC * NS
BW = 128
L = 16
NG = BW // L


def _body(xt_hbm, emb4_hbm, post_hbm, out_hbm, tok_v, idx4_a, idx4_b, rows_a,
          rows_b, xp_v, pos_v, sem_a, sem_b):
    cid = lax.axis_index("c")
    sid = lax.axis_index("s")
    wid = sid * NC + cid

    seq = xt_hbm.shape[0]
    hid = pos_v.shape[0]
    b0 = wid * BW

    pltpu.sync_copy(post_hbm, pos_v)
    pltpu.sync_copy(xt_hbm.at[pl.ds(0, seq), pl.ds(b0, BW)], tok_v)

    iota = lax.iota(jnp.int32, L)
    ridx = [iota + g * L for g in range(NG)]

    def fire(s, idx_buf, rows_buf, sem):
        for g in range(NG):
            tv = tok_v[s, pl.ds(g * L, L)]
            idx_buf[pl.ds(g * L, L)] = lax.shift_right_logical(tv, 2)
        return pltpu.async_copy(emb4_hbm.at[idx_buf], rows_buf, sem)

    def compute(s, rows_buf):
        sv = jnp.full((L,), s, jnp.int32)
        cbase = []
        for g in range(NG):
            tv = tok_v[s, pl.ds(g * L, L)]
            cbase.append(lax.shift_left(jnp.bitwise_and(tv, 3), 5))

        @plsc.parallel_loop(0, hid, step=1, unroll=4)
        def _h(h):
            hv = jnp.full((L,), h, jnp.int32)
            p = plsc.load_gather(pos_v, [hv, sv])
            for g in range(NG):
                vec = plsc.load_gather(rows_buf, [ridx[g], cbase[g] + hv])
                xp_v[h, pl.ds(g * L, L)] = vec + p

        pltpu.sync_copy(xp_v, out_hbm.at[s, pl.ds(0, hid), pl.ds(b0, BW)])

    cp0 = fire(0, idx4_a, rows_a, sem_a)

    def sloop(i, carry):
        s0 = 2 * i
        s1 = s0 + 1
        s2 = jnp.minimum(s0 + 2, seq - 1)
        cp_b = fire(s1, idx4_b, rows_b, sem_b)
        pltpu.make_async_copy(emb4_hbm.at[idx4_a], rows_a, sem_a).wait()
        compute(s0, rows_a)
        cp_a = fire(s2, idx4_a, rows_a, sem_a)
        pltpu.make_async_copy(emb4_hbm.at[idx4_b], rows_b, sem_b).wait()
        compute(s1, rows_b)
        return carry

    lax.fori_loop(0, seq // 2, sloop, 0)
    pltpu.make_async_copy(emb4_hbm.at[idx4_a], rows_a, sem_a).wait()


def _prep(embt_hbm, out4_hbm, in_v, out_v, tin_v):
    cid = lax.axis_index("c")
    sid = lax.axis_index("s")
    wid = sid * NC + cid

    vocab = embt_hbm.shape[1]
    n_full = vocab // 512
    tail = vocab - n_full * 512
    nw = (n_full - wid + NW - 1) // NW

    iota = lax.iota(jnp.int32, L)
    iotb = iota + L

    def kloop(k, carry):
        c = wid + NW * k
        pltpu.sync_copy(embt_hbm.at[pl.ds(0, 32), pl.ds(c * 512, 512)], in_v)

        @plsc.parallel_loop(0, 512, step=1, unroll=4)
        def _j(j):
            jv = jnp.full((L,), j, jnp.int32)
            r = lax.shift_right_logical(j, 2)
            cst = lax.shift_left(jnp.bitwise_and(j, 3), 5)
            out_v[r, pl.ds(cst, L)] = plsc.load_gather(in_v, [iota, jv])
            out_v[r, pl.ds(cst + L, L)] = plsc.load_gather(in_v, [iotb, jv])

        pltpu.sync_copy(out_v, out4_hbm.at[pl.ds(c * 128, 128)])
        return carry

    lax.fori_loop(0, nw, kloop, 0)

    if tail:
        @pl.when(wid == 1)
        def _tail():
            base = n_full * 512
            pltpu.sync_copy(
                embt_hbm.at[pl.ds(0, 32), pl.ds(base, tail)],
                tin_v)

            @plsc.parallel_loop(0, tail, step=1, unroll=4)
            def _j(j):
                jv = jnp.full((L,), j, jnp.int32)
                r = lax.shift_right_logical(j, 2)
                cst = lax.shift_left(jnp.bitwise_and(j, 3), 5)
                out_v[r, pl.ds(cst, L)] = plsc.load_gather(tin_v, [iota, jv])
                out_v[r, pl.ds(cst + L, L)] = plsc.load_gather(tin_v, [iotb, jv])

            pltpu.sync_copy(
                out_v.at[pl.ds(0, tail // 4)],
                out4_hbm.at[pl.ds(base // 4, tail // 4)])


def kernel(x, emb_table, pos_table):
    batch, seq_len = x.shape
    hid = emb_table.shape[1]

    xt = jnp.transpose(x)
    embt = jnp.transpose(emb_table)
    post = jnp.transpose(pos_table)

    prep = pl.kernel(
        _prep,
        out_type=jax.ShapeDtypeStruct(
            (emb_table.shape[0] * hid // 128, 128), jnp.float32),
        mesh=plsc.VectorSubcoreMesh(core_axis_name="c", subcore_axis_name="s"),
        scratch_types=[
            pltpu.VMEM((hid, 512), jnp.float32),
            pltpu.VMEM((128, 128), jnp.float32),
            pltpu.VMEM((hid, 64), jnp.float32),
        ],
        compiler_params=pltpu.CompilerParams(
            use_tc_tiling_on_sc=True, needs_layout_passes=False),
    )
    emb4 = prep(embt)

    call = pl.kernel(
        _body,
        out_type=jax.ShapeDtypeStruct((seq_len, hid, batch), jnp.float32),
        mesh=plsc.VectorSubcoreMesh(core_axis_name="c", subcore_axis_name="s"),
        scratch_types=[
            pltpu.VMEM((seq_len, BW), jnp.int32),
            pltpu.VMEM((BW,), jnp.int32),
            pltpu.VMEM((BW,), jnp.int32),
            pltpu.VMEM((BW, 128), jnp.float32),
            pltpu.VMEM((BW, 128), jnp.float32),
            pltpu.VMEM((hid, BW), jnp.float32),
            pltpu.VMEM((hid, seq_len), jnp.float32),
            pltpu.SemaphoreType.DMA,
            pltpu.SemaphoreType.DMA,
        ],
        compiler_params=pltpu.CompilerParams(
            use_tc_tiling_on_sc=True, needs_layout_passes=False),
    )
    out = call(xt, emb4, post)
    return jnp.transpose(out, (2, 0, 1))


# speedup vs baseline: 1.1411x; 1.1411x over previous
import functools

import jax
import jax.numpy as jnp
from jax import lax
from jax.experimental import pallas as pl
from jax.experimental.pallas import tpu as pltpu
from jax.experimental.pallas import tpu_sc as plsc

NC = 2
NS = 16
NW = NC * NS
BW = 128
L = 16
NG = BW // L


def _body(xt_hbm, emb4_hbm, post_hbm, out_hbm, tok_v, idx4_a, idx4_b, rows_a,
          rows_b, xp_v, pos_v, sem_a, sem_b):
    cid = lax.axis_index("c")
    sid = lax.axis_index("s")
    wid = sid * NC + cid

    seq = xt_hbm.shape[0]
    hid = pos_v.shape[0]
    b0 = wid * BW

    pltpu.sync_copy(post_hbm, pos_v)
    pltpu.sync_copy(xt_hbm.at[pl.ds(0, seq), pl.ds(b0, BW)], tok_v)

    iota = lax.iota(jnp.int32, L)
    ridx = [iota + g * L for g in range(NG)]

    def fire(s, idx_buf, rows_buf, sem):
        for g in range(NG):
            tv = tok_v[s, pl.ds(g * L, L)]
            idx_buf[pl.ds(g * L, L)] = lax.shift_right_logical(tv, 2)
        return pltpu.async_copy(emb4_hbm.at[idx_buf], rows_buf, sem)

    def compute(s, rows_buf):
        sv = jnp.full((L,), s, jnp.int32)
        cbase = []
        for g in range(NG):
            tv = tok_v[s, pl.ds(g * L, L)]
            cbase.append(lax.shift_left(jnp.bitwise_and(tv, 3), 5))

        @plsc.parallel_loop(0, hid, step=1, unroll=4)
        def _h(h):
            hv = jnp.full((L,), h, jnp.int32)
            p = plsc.load_gather(pos_v, [hv, sv])
            for g in range(NG):
                vec = plsc.load_gather(rows_buf, [ridx[g], cbase[g] + hv])
                xp_v[h, pl.ds(g * L, L)] = vec + p

        pltpu.sync_copy(xp_v, out_hbm.at[s, pl.ds(0, hid), pl.ds(b0, BW)])

    cp0 = fire(0, idx4_a, rows_a, sem_a)

    def sloop(i, carry):
        s0 = 2 * i
        s1 = s0 + 1
        s2 = jnp.minimum(s0 + 2, seq - 1)
        cp_b = fire(s1, idx4_b, rows_b, sem_b)
        pltpu.make_async_copy(emb4_hbm.at[idx4_a], rows_a, sem_a).wait()
        compute(s0, rows_a)
        cp_a = fire(s2, idx4_a, rows_a, sem_a)
        pltpu.make_async_copy(emb4_hbm.at[idx4_b], rows_b, sem_b).wait()
        compute(s1, rows_b)
        return carry

    lax.fori_loop(0, seq // 2, sloop, 0)
    pltpu.make_async_copy(emb4_hbm.at[idx4_a], rows_a, sem_a).wait()


def _prep(embt_hbm, out4_hbm, in_a, in_b, out_a, out_b, tin_v,
          sem_ia, sem_ib, sem_oa, sem_ob):
    cid = lax.axis_index("c")
    sid = lax.axis_index("s")
    wid = sid * NC + cid

    vocab = embt_hbm.shape[1]
    n_full = vocab // 512
    tail = vocab - n_full * 512
    nw = (n_full - wid + NW - 1) // NW
    last = wid + NW * (nw - 1)
    npair = (n_full + NW - 1) // NW // 2 + (((n_full + NW - 1) // NW) % 2)

    iota = lax.iota(jnp.int32, L)
    iotb = iota + L

    def fire_in(c, buf, sem):
        return pltpu.async_copy(
            embt_hbm.at[pl.ds(0, 32), pl.ds(c * 512, 512)], buf, sem)

    def wait_in(buf, sem):
        pltpu.make_async_copy(
            embt_hbm.at[pl.ds(0, 32), pl.ds(0, 512)], buf, sem).wait()

    def wait_out(buf, sem):
        pltpu.make_async_copy(
            out4_hbm.at[pl.ds(0, 128)], buf, sem).wait()

    def xpose(in_buf, out_buf):
        @plsc.parallel_loop(0, 128, step=1, unroll=4)
        def _r(r):
            bv = jnp.full((L,), lax.shift_left(r, 2), jnp.int32)
            for q in range(4):
                jv = bv + q
                out_buf[r, pl.ds(q * 32, L)] = plsc.load_gather(
                    in_buf, [iota, jv])
                out_buf[r, pl.ds(q * 32 + L, L)] = plsc.load_gather(
                    in_buf, [iotb, jv])

    cp0 = fire_in(jnp.minimum(wid, last), in_a, sem_ia)

    def kloop(i, carry):
        ca = jnp.minimum(wid + NW * (2 * i), last)
        cb = jnp.minimum(wid + NW * (2 * i + 1), last)
        cn = jnp.minimum(wid + NW * (2 * i + 2), last)
        wait_in(in_a, sem_ia)
        fire_in(cb, in_b, sem_ib)

        @pl.when(i >= 1)
        def _():
            wait_out(out_a, sem_oa)
        xpose(in_a, out_a)
        pltpu.async_copy(out_a, out4_hbm.at[pl.ds(ca * 128, 128)], sem_oa)

        wait_in(in_b, sem_ib)
        fire_in(cn, in_a, sem_ia)

        @pl.when(i >= 1)
        def _():
            wait_out(out_b, sem_ob)
        xpose(in_b, out_b)
        pltpu.async_copy(out_b, out4_hbm.at[pl.ds(cb * 128, 128)], sem_ob)
        return carry

    lax.fori_loop(0, npair, kloop, 0)
    wait_in(in_a, sem_ia)
    wait_out(out_a, sem_oa)
    wait_out(out_b, sem_ob)

    if tail:
        @pl.when(wid == 1)
        def _tail():
            base = n_full * 512
            pltpu.sync_copy(
                embt_hbm.at[pl.ds(0, 32), pl.ds(base, tail)],
                tin_v)

            @plsc.parallel_loop(0, tail // 4, step=1, unroll=4)
            def _r(r):
                bv = jnp.full((L,), lax.shift_left(r, 2), jnp.int32)
                for q in range(4):
                    jv = bv + q
                    out_a[r, pl.ds(q * 32, L)] = plsc.load_gather(
                        tin_v, [iota, jv])
                    out_a[r, pl.ds(q * 32 + L, L)] = plsc.load_gather(
                        tin_v, [iotb, jv])

            pltpu.sync_copy(
                out_a.at[pl.ds(0, tail // 4)],
                out4_hbm.at[pl.ds(base // 4, tail // 4)])


def kernel(x, emb_table, pos_table):
    batch, seq_len = x.shape
    hid = emb_table.shape[1]

    xt = jnp.transpose(x)
    embt = jnp.transpose(emb_table)
    post = jnp.transpose(pos_table)

    prep = pl.kernel(
        _prep,
        out_type=jax.ShapeDtypeStruct(
            (emb_table.shape[0] * hid // 128, 128), jnp.float32),
        mesh=plsc.VectorSubcoreMesh(core_axis_name="c", subcore_axis_name="s"),
        scratch_types=[
            pltpu.VMEM((hid, 512), jnp.float32),
            pltpu.VMEM((hid, 512), jnp.float32),
            pltpu.VMEM((128, 128), jnp.float32),
            pltpu.VMEM((128, 128), jnp.float32),
            pltpu.VMEM((hid, 64), jnp.float32),
            pltpu.SemaphoreType.DMA,
            pltpu.SemaphoreType.DMA,
            pltpu.SemaphoreType.DMA,
            pltpu.SemaphoreType.DMA,
        ],
        compiler_params=pltpu.CompilerParams(
            use_tc_tiling_on_sc=True, needs_layout_passes=False),
    )
    emb4 = prep(embt)

    call = pl.kernel(
        _body,
        out_type=jax.ShapeDtypeStruct((seq_len, hid, batch), jnp.float32),
        mesh=plsc.VectorSubcoreMesh(core_axis_name="c", subcore_axis_name="s"),
        scratch_types=[
            pltpu.VMEM((seq_len, BW), jnp.int32),
            pltpu.VMEM((BW,), jnp.int32),
            pltpu.VMEM((BW,), jnp.int32),
            pltpu.VMEM((BW, 128), jnp.float32),
            pltpu.VMEM((BW, 128), jnp.float32),
            pltpu.VMEM((hid, BW), jnp.float32),
            pltpu.VMEM((hid, seq_len), jnp.float32),
            pltpu.SemaphoreType.DMA,
            pltpu.SemaphoreType.DMA,
        ],
        compiler_params=pltpu.CompilerParams(
            use_tc_tiling_on_sc=True, needs_layout_passes=False),
    )
    out = call(xt, emb4, post)
    return jnp.transpose(out, (2, 0, 1))


# speedup vs baseline: 1.1561x; 1.0132x over previous
import functools

import jax
import jax.numpy as jnp
from jax import lax
from jax.experimental import pallas as pl
from jax.experimental.pallas import tpu as pltpu
from jax.experimental.pallas import tpu_sc as plsc

NC = 2
NS = 16
NW = NC * NS
BW = 128
L = 16
NG = BW // L


def _body(xt_hbm, emb4_hbm, post_hbm, out_hbm, tok_v, idx4_a, idx4_b, rows_a,
          rows_b, xp_v, pos_v, sem_a, sem_b):
    cid = lax.axis_index("c")
    sid = lax.axis_index("s")
    wid = sid * NC + cid

    seq = xt_hbm.shape[0]
    hid = pos_v.shape[0]
    b0 = wid * BW

    pltpu.sync_copy(post_hbm, pos_v)
    pltpu.sync_copy(xt_hbm.at[pl.ds(0, seq), pl.ds(b0, BW)], tok_v)

    iota = lax.iota(jnp.int32, L)
    ridx = [iota + g * L for g in range(NG)]

    def fire(s, idx_buf, rows_buf, sem):
        for g in range(NG):
            tv = tok_v[s, pl.ds(g * L, L)]
            idx_buf[pl.ds(g * L, L)] = lax.shift_right_logical(tv, 2)
        return pltpu.async_copy(emb4_hbm.at[idx_buf], rows_buf, sem)

    def compute(s, rows_buf):
        sv = jnp.full((L,), s, jnp.int32)
        cbase = []
        for g in range(NG):
            tv = tok_v[s, pl.ds(g * L, L)]
            cbase.append(lax.shift_left(jnp.bitwise_and(tv, 3), 5))

        @plsc.parallel_loop(0, hid, step=1, unroll=4)
        def _h(h):
            hv = jnp.full((L,), h, jnp.int32)
            p = plsc.load_gather(pos_v, [hv, sv])
            for g in range(NG):
                vec = plsc.load_gather(rows_buf, [ridx[g], cbase[g] + hv])
                xp_v[h, pl.ds(g * L, L)] = vec + p

        pltpu.sync_copy(xp_v, out_hbm.at[s, pl.ds(0, hid), pl.ds(b0, BW)])

    cp0 = fire(0, idx4_a, rows_a, sem_a)

    def sloop(i, carry):
        s0 = 2 * i
        s1 = s0 + 1
        s2 = jnp.minimum(s0 + 2, seq - 1)
        cp_b = fire(s1, idx4_b, rows_b, sem_b)
        pltpu.make_async_copy(emb4_hbm.at[idx4_a], rows_a, sem_a).wait()
        compute(s0, rows_a)
        cp_a = fire(s2, idx4_a, rows_a, sem_a)
        pltpu.make_async_copy(emb4_hbm.at[idx4_b], rows_b, sem_b).wait()
        compute(s1, rows_b)
        return carry

    lax.fori_loop(0, seq // 2, sloop, 0)
    pltpu.make_async_copy(emb4_hbm.at[idx4_a], rows_a, sem_a).wait()


def _prep(embt_hbm, out4_hbm, in_a, in_b, out_a, out_b, tin_v,
          sem_ia, sem_ib, sem_oa, sem_ob):
    cid = lax.axis_index("c")
    sid = lax.axis_index("s")
    wid = sid * NC + cid

    vocab = embt_hbm.shape[1]
    n_full = vocab // 512
    tail = vocab - n_full * 512
    nw = (n_full - wid + NW - 1) // NW
    last = wid + NW * (nw - 1)
    npair = (n_full + NW - 1) // NW // 2 + (((n_full + NW - 1) // NW) % 2)

    iota = lax.iota(jnp.int32, L)
    t0 = lax.shift_left(lax.shift_right_logical(iota, 3), 2)
    t1 = t0 + 8
    hrow = jnp.bitwise_and(iota, 7)

    def fire_in(c, buf, sem):
        for tr in range(4):
            for tc in range(4):
                pltpu.async_copy(
                    embt_hbm.at[pl.ds(tr * 8, 8),
                                pl.ds(c * 512 + tc * 128, 128)],
                    buf.at[tr * 4 + tc], sem)

    def wait_in(buf, sem):
        for t in range(16):
            pltpu.make_async_copy(
                embt_hbm.at[pl.ds(0, 8), pl.ds(0, 128)],
                buf.at[t], sem).wait()

    def wait_out(buf, sem):
        pltpu.make_async_copy(
            out4_hbm.at[pl.ds(0, 128)], buf, sem).wait()

    def xpose(in_buf, out_buf):
        @plsc.parallel_loop(0, 128, step=1, unroll=4)
        def _r(r):
            for q in range(4):
                jb = lax.shift_left(r, 2) + q
                tv = jnp.full((L,), lax.shift_right_logical(jb, 7), jnp.int32)
                jv = jnp.full((L,), jnp.bitwise_and(jb, 127), jnp.int32)
                out_buf[r, pl.ds(q * 32, L)] = plsc.load_gather(
                    in_buf, [t0 + tv, hrow, jv])
                out_buf[r, pl.ds(q * 32 + L, L)] = plsc.load_gather(
                    in_buf, [t1 + tv, hrow, jv])

    cp0 = fire_in(jnp.minimum(wid, last), in_a, sem_ia)

    def kloop(i, carry):
        ca = jnp.minimum(wid + NW * (2 * i), last)
        cb = jnp.minimum(wid + NW * (2 * i + 1), last)
        cn = jnp.minimum(wid + NW * (2 * i + 2), last)
        wait_in(in_a, sem_ia)
        fire_in(cb, in_b, sem_ib)

        @pl.when(i >= 1)
        def _():
            wait_out(out_a, sem_oa)
        xpose(in_a, out_a)
        pltpu.async_copy(out_a, out4_hbm.at[pl.ds(ca * 128, 128)], sem_oa)

        wait_in(in_b, sem_ib)
        fire_in(cn, in_a, sem_ia)

        @pl.when(i >= 1)
        def _():
            wait_out(out_b, sem_ob)
        xpose(in_b, out_b)
        pltpu.async_copy(out_b, out4_hbm.at[pl.ds(cb * 128, 128)], sem_ob)
        return carry

    lax.fori_loop(0, npair, kloop, 0)
    wait_in(in_a, sem_ia)
    wait_out(out_a, sem_oa)
    wait_out(out_b, sem_ob)

    if tail:
        @pl.when(wid == 1)
        def _tail():
            base = n_full * 512
            for tr in range(4):
                pltpu.sync_copy(
                    embt_hbm.at[pl.ds(tr * 8, 8), pl.ds(base, tail)],
                    tin_v.at[tr])
            tt0 = lax.shift_right_logical(iota, 3)
            tt1 = tt0 + 2

            @plsc.parallel_loop(0, tail // 4, step=1, unroll=4)
            def _r(r):
                for q in range(4):
                    jv = jnp.full((L,), lax.shift_left(r, 2) + q, jnp.int32)
                    out_a[r, pl.ds(q * 32, L)] = plsc.load_gather(
                        tin_v, [tt0, hrow, jv])
                    out_a[r, pl.ds(q * 32 + L, L)] = plsc.load_gather(
                        tin_v, [tt1, hrow, jv])

            pltpu.sync_copy(
                out_a.at[pl.ds(0, tail // 4)],
                out4_hbm.at[pl.ds(base // 4, tail // 4)])


def kernel(x, emb_table, pos_table):
    batch, seq_len = x.shape
    hid = emb_table.shape[1]

    xt = jnp.transpose(x)
    embt = jnp.transpose(emb_table)
    post = jnp.transpose(pos_table)

    prep = pl.kernel(
        _prep,
        out_type=jax.ShapeDtypeStruct(
            (emb_table.shape[0] * hid // 128, 128), jnp.float32),
        mesh=plsc.VectorSubcoreMesh(core_axis_name="c", subcore_axis_name="s"),
        scratch_types=[
            pltpu.VMEM((16, 8, 128), jnp.float32),
            pltpu.VMEM((16, 8, 128), jnp.float32),
            pltpu.VMEM((128, 128), jnp.float32),
            pltpu.VMEM((128, 128), jnp.float32),
            pltpu.VMEM((4, 8, 64), jnp.float32),
            pltpu.SemaphoreType.DMA,
            pltpu.SemaphoreType.DMA,
            pltpu.SemaphoreType.DMA,
            pltpu.SemaphoreType.DMA,
        ],
        compiler_params=pltpu.CompilerParams(
            use_tc_tiling_on_sc=True, needs_layout_passes=False),
    )
    emb4 = prep(embt)

    call = pl.kernel(
        _body,
        out_type=jax.ShapeDtypeStruct((seq_len, hid, batch), jnp.float32),
        mesh=plsc.VectorSubcoreMesh(core_axis_name="c", subcore_axis_name="s"),
        scratch_types=[
            pltpu.VMEM((seq_len, BW), jnp.int32),
            pltpu.VMEM((BW,), jnp.int32),
            pltpu.VMEM((BW,), jnp.int32),
            pltpu.VMEM((BW, 128), jnp.float32),
            pltpu.VMEM((BW, 128), jnp.float32),
            pltpu.VMEM((hid, BW), jnp.float32),
            pltpu.VMEM((hid, seq_len), jnp.float32),
            pltpu.SemaphoreType.DMA,
            pltpu.SemaphoreType.DMA,
        ],
        compiler_params=pltpu.CompilerParams(
            use_tc_tiling_on_sc=True, needs_layout_passes=False),
    )
    out = call(xt, emb4, post)
    return jnp.transpose(out, (2, 0, 1))


# speedup vs baseline: 1.2170x; 1.0527x over previous
import functools

import jax
import jax.numpy as jnp
from jax import lax
from jax.experimental import pallas as pl
from jax.experimental.pallas import tpu as pltpu
from jax.experimental.pallas import tpu_sc as plsc

NC = 2
NS = 16
NW = NC * NS
BW = 128
L = 16
NG = BW // L


def _body(xt_hbm, emb4_hbm, post_hbm, out_hbm, tok_v, idx4_a, idx4_b, rows_a,
          rows_b, xp_a, xp_b, pos_v, sem_a, sem_b, sem_oa, sem_ob):
    cid = lax.axis_index("c")
    sid = lax.axis_index("s")
    wid = sid * NC + cid

    seq = xt_hbm.shape[0]
    hid = pos_v.shape[0]
    b0 = wid * BW

    pltpu.sync_copy(post_hbm, pos_v)
    pltpu.sync_copy(xt_hbm.at[pl.ds(0, seq), pl.ds(b0, BW)], tok_v)

    iota = lax.iota(jnp.int32, L)
    ridx = [iota + g * L for g in range(NG)]

    def fire(s, idx_buf, rows_buf, sem):
        for g in range(NG):
            tv = tok_v[s, pl.ds(g * L, L)]
            idx_buf[pl.ds(g * L, L)] = lax.shift_right_logical(tv, 2)
        return pltpu.async_copy(emb4_hbm.at[idx_buf], rows_buf, sem)

    def compute(s, rows_buf, xp_buf, sem_o):
        @pl.when(s >= 2)
        def _():
            pltpu.make_async_copy(
                out_hbm.at[0, pl.ds(0, hid), pl.ds(b0, BW)],
                xp_buf, sem_o).wait()

        sv = jnp.full((L,), s, jnp.int32)
        cbase = []
        for g in range(NG):
            tv = tok_v[s, pl.ds(g * L, L)]
            cbase.append(lax.shift_left(jnp.bitwise_and(tv, 3), 5))

        @plsc.parallel_loop(0, hid, step=1, unroll=4)
        def _h(h):
            hv = jnp.full((L,), h, jnp.int32)
            p = plsc.load_gather(pos_v, [hv, sv])
            for g in range(NG):
                vec = plsc.load_gather(rows_buf, [ridx[g], cbase[g] + hv])
                xp_buf[h, pl.ds(g * L, L)] = vec + p

        pltpu.async_copy(
            xp_buf, out_hbm.at[s, pl.ds(0, hid), pl.ds(b0, BW)], sem_o)

    cp0 = fire(0, idx4_a, rows_a, sem_a)

    def sloop(i, carry):
        s0 = 2 * i
        s1 = s0 + 1
        s2 = jnp.minimum(s0 + 2, seq - 1)
        cp_b = fire(s1, idx4_b, rows_b, sem_b)
        pltpu.make_async_copy(emb4_hbm.at[idx4_a], rows_a, sem_a).wait()
        compute(s0, rows_a, xp_a, sem_oa)
        cp_a = fire(s2, idx4_a, rows_a, sem_a)
        pltpu.make_async_copy(emb4_hbm.at[idx4_b], rows_b, sem_b).wait()
        compute(s1, rows_b, xp_b, sem_ob)
        return carry

    lax.fori_loop(0, seq // 2, sloop, 0)
    pltpu.make_async_copy(emb4_hbm.at[idx4_a], rows_a, sem_a).wait()
    for xp_buf, sem_o in ((xp_a, sem_oa), (xp_b, sem_ob)):
        pltpu.make_async_copy(
            out_hbm.at[0, pl.ds(0, hid), pl.ds(b0, BW)],
            xp_buf, sem_o).wait()


def _prep(embt_hbm, out4_hbm, in_a, in_b, out_a, out_b, tin_v,
          sem_ia, sem_ib, sem_oa, sem_ob):
    cid = lax.axis_index("c")
    sid = lax.axis_index("s")
    wid = sid * NC + cid

    vocab = embt_hbm.shape[1]
    n_full = vocab // 512
    tail = vocab - n_full * 512
    nw = (n_full - wid + NW - 1) // NW
    last = wid + NW * (nw - 1)
    npair = (n_full + NW - 1) // NW // 2 + (((n_full + NW - 1) // NW) % 2)

    iota = lax.iota(jnp.int32, L)
    t0 = lax.shift_left(lax.shift_right_logical(iota, 3), 2)
    t1 = t0 + 8
    hrow = jnp.bitwise_and(iota, 7)

    def fire_in(c, buf, sem):
        for tr in range(4):
            for tc in range(4):
                pltpu.async_copy(
                    embt_hbm.at[pl.ds(tr * 8, 8),
                                pl.ds(c * 512 + tc * 128, 128)],
                    buf.at[tr * 4 + tc], sem)

    def wait_in(buf, sem):
        for t in range(16):
            pltpu.make_async_copy(
                embt_hbm.at[pl.ds(0, 8), pl.ds(0, 128)],
                buf.at[t], sem).wait()

    def wait_out(buf, sem):
        pltpu.make_async_copy(
            out4_hbm.at[pl.ds(0, 128)], buf, sem).wait()

    def xpose(in_buf, out_buf):
        @plsc.parallel_loop(0, 128, step=1, unroll=4)
        def _r(r):
            for q in range(4):
                jb = lax.shift_left(r, 2) + q
                tv = jnp.full((L,), lax.shift_right_logical(jb, 7), jnp.int32)
                jv = jnp.full((L,), jnp.bitwise_and(jb, 127), jnp.int32)
                out_buf[r, pl.ds(q * 32, L)] = plsc.load_gather(
                    in_buf, [t0 + tv, hrow, jv])
                out_buf[r, pl.ds(q * 32 + L, L)] = plsc.load_gather(
                    in_buf, [t1 + tv, hrow, jv])

    cp0 = fire_in(jnp.minimum(wid, last), in_a, sem_ia)

    def kloop(i, carry):
        ca = jnp.minimum(wid + NW * (2 * i), last)
        cb = jnp.minimum(wid + NW * (2 * i + 1), last)
        cn = jnp.minimum(wid + NW * (2 * i + 2), last)
        wait_in(in_a, sem_ia)
        fire_in(cb, in_b, sem_ib)

        @pl.when(i >= 1)
        def _():
            wait_out(out_a, sem_oa)
        xpose(in_a, out_a)
        pltpu.async_copy(out_a, out4_hbm.at[pl.ds(ca * 128, 128)], sem_oa)

        wait_in(in_b, sem_ib)
        fire_in(cn, in_a, sem_ia)

        @pl.when(i >= 1)
        def _():
            wait_out(out_b, sem_ob)
        xpose(in_b, out_b)
        pltpu.async_copy(out_b, out4_hbm.at[pl.ds(cb * 128, 128)], sem_ob)
        return carry

    lax.fori_loop(0, npair, kloop, 0)
    wait_in(in_a, sem_ia)
    wait_out(out_a, sem_oa)
    wait_out(out_b, sem_ob)

    if tail:
        @pl.when(wid == 1)
        def _tail():
            base = n_full * 512
            for tr in range(4):
                pltpu.sync_copy(
                    embt_hbm.at[pl.ds(tr * 8, 8), pl.ds(base, tail)],
                    tin_v.at[tr])
            tt0 = lax.shift_right_logical(iota, 3)
            tt1 = tt0 + 2

            @plsc.parallel_loop(0, tail // 4, step=1, unroll=4)
            def _r(r):
                for q in range(4):
                    jv = jnp.full((L,), lax.shift_left(r, 2) + q, jnp.int32)
                    out_a[r, pl.ds(q * 32, L)] = plsc.load_gather(
                        tin_v, [tt0, hrow, jv])
                    out_a[r, pl.ds(q * 32 + L, L)] = plsc.load_gather(
                        tin_v, [tt1, hrow, jv])

            pltpu.sync_copy(
                out_a.at[pl.ds(0, tail // 4)],
                out4_hbm.at[pl.ds(base // 4, tail // 4)])


def kernel(x, emb_table, pos_table):
    batch, seq_len = x.shape
    hid = emb_table.shape[1]

    xt = jnp.transpose(x)
    embt = jnp.transpose(emb_table)
    post = jnp.transpose(pos_table)

    prep = pl.kernel(
        _prep,
        out_type=jax.ShapeDtypeStruct(
            (emb_table.shape[0] * hid // 128, 128), jnp.float32),
        mesh=plsc.VectorSubcoreMesh(core_axis_name="c", subcore_axis_name="s"),
        scratch_types=[
            pltpu.VMEM((16, 8, 128), jnp.float32),
            pltpu.VMEM((16, 8, 128), jnp.float32),
            pltpu.VMEM((128, 128), jnp.float32),
            pltpu.VMEM((128, 128), jnp.float32),
            pltpu.VMEM((4, 8, 64), jnp.float32),
            pltpu.SemaphoreType.DMA,
            pltpu.SemaphoreType.DMA,
            pltpu.SemaphoreType.DMA,
            pltpu.SemaphoreType.DMA,
        ],
        compiler_params=pltpu.CompilerParams(
            use_tc_tiling_on_sc=True, needs_layout_passes=False),
    )
    emb4 = prep(embt)

    call = pl.kernel(
        _body,
        out_type=jax.ShapeDtypeStruct((seq_len, hid, batch), jnp.float32),
        mesh=plsc.VectorSubcoreMesh(core_axis_name="c", subcore_axis_name="s"),
        scratch_types=[
            pltpu.VMEM((seq_len, BW), jnp.int32),
            pltpu.VMEM((BW,), jnp.int32),
            pltpu.VMEM((BW,), jnp.int32),
            pltpu.VMEM((BW, 128), jnp.float32),
            pltpu.VMEM((BW, 128), jnp.float32),
            pltpu.VMEM((hid, BW), jnp.float32),
            pltpu.VMEM((hid, BW), jnp.float32),
            pltpu.VMEM((hid, seq_len), jnp.float32),
            pltpu.SemaphoreType.DMA,
            pltpu.SemaphoreType.DMA,
            pltpu.SemaphoreType.DMA,
            pltpu.SemaphoreType.DMA,
        ],
        compiler_params=pltpu.CompilerParams(
            use_tc_tiling_on_sc=True, needs_layout_passes=False),
    )
    out = call(xt, emb4, post)
    return jnp.transpose(out, (2, 0, 1))
